# Initial kernel scaffold; baseline (speedup 1.0000x reference)
import jax
import jax.numpy as jnp
from jax.experimental import pallas as pl


def kernel(x0, x1, adj0, adj1, W1_0, b1_0, W2_0, b2_0, W1_1, b1_1, W2_1, b2_1, Wfuse, bfuse):
    raise NotImplementedError("write your pallas kernel here")


# speedup vs baseline: 4.2898x; 4.2898x over previous
import jax
import jax.numpy as jnp
from jax.experimental import pallas as pl

_N = 1024
_K_NN = 20
_T_ITERS = 20


def _mm(a, b):
    return jax.lax.dot_general(
        a, b, (((1,), (0,)), ((), ())),
        precision=jax.lax.Precision.DEFAULT,
        preferred_element_type=jnp.float32)


def _mm_t(a, b):
    return jax.lax.dot_general(
        a, b, (((1,), (1,)), ((), ())),
        precision=jax.lax.Precision.DEFAULT,
        preferred_element_type=jnp.float32)


def _snf_graph(d):
    sq = jnp.sum(d * d, axis=1, keepdims=True)
    g = _mm_t(d, d)
    d2 = jnp.maximum(sq + jnp.transpose(sq) - 2.0 * g, 0.0)
    dist = jnp.sqrt(d2)
    sigma = (jnp.sum(dist) / float(_N * (_N - 1))) * 0.5
    w = jnp.exp(-(dist * dist) / (2.0 * (sigma * sigma)))
    dcol = jnp.sum(w, axis=1, keepdims=True)
    dinv = 1.0 / jnp.sqrt(dcol)
    p = dinv * w * jnp.transpose(dinv)

    def strip_max(_, cur):
        m = jnp.max(cur, axis=1, keepdims=True)
        return jnp.where(cur >= m, -jnp.inf, cur)

    stripped = jax.lax.fori_loop(0, _K_NN - 1, strip_max, p)
    thr = jnp.max(stripped, axis=1, keepdims=True)
    wk = jnp.where(p >= thr, p, 0.0)
    return (wk + jnp.transpose(wk)) * 0.5


def _body(x0_ref, x1_ref, d0_ref, d1_ref,
          w10_ref, b10_ref, w20_ref, b20_ref,
          w11_ref, b11_ref, w21_ref, b21_ref,
          wfa_ref, wfb_ref, bf_ref, out_ref):
    ws0 = _snf_graph(d0_ref[:])
    ws1 = _snf_graph(d1_ref[:])
    wf = (ws0 + ws1) * 0.5

    def diff_body(_, wf):
        wn = (_mm(_mm(ws0, wf), ws0) + _mm(_mm(ws1, wf), ws1)) * 0.5
        dcol = jnp.sum(wn, axis=1, keepdims=True)
        dinv = 1.0 / jnp.sqrt(dcol)
        return dinv * wn * jnp.transpose(dinv)

    wf = jax.lax.fori_loop(0, _T_ITERS, diff_body, wf)

    h0 = jax.nn.relu(_mm(wf, _mm(x0_ref[:], w10_ref[:])) + b10_ref[:])
    h0 = _mm(wf, _mm(h0, w20_ref[:])) + b20_ref[:]
    h1 = jax.nn.relu(_mm(wf, _mm(x1_ref[:], w11_ref[:])) + b11_ref[:])
    h1 = _mm(wf, _mm(h1, w21_ref[:])) + b21_ref[:]

    out_ref[:] = _mm(h0, wfa_ref[:]) + _mm(h1, wfb_ref[:]) + bf_ref[:]


def kernel(x0, x1, adj0, adj1, W1_0, b1_0, W2_0, b2_0,
           W1_1, b1_1, W2_1, b2_1, Wfuse, bfuse):
    hidden = W2_0.shape[1]
    return pl.pallas_call(
        _body,
        out_shape=jax.ShapeDtypeStruct((x0.shape[0], Wfuse.shape[1]),
                                       jnp.float32),
    )(x0, x1, adj0, adj1,
      W1_0, b1_0.reshape(1, -1), W2_0, b2_0.reshape(1, -1),
      W1_1, b1_1.reshape(1, -1), W2_1, b2_1.reshape(1, -1),
      Wfuse[:hidden], Wfuse[hidden:], bfuse.reshape(1, -1))


# speedup vs baseline: 4.3033x; 1.0031x over previous
import jax
import jax.numpy as jnp
from jax.experimental import pallas as pl

_N = 1024
_K_NN = 20
_T_ITERS = 20


def _mm(a, b):
    return jax.lax.dot_general(
        a, b, (((1,), (0,)), ((), ())),
        precision=jax.lax.Precision.DEFAULT,
        preferred_element_type=jnp.float32)


def _mm_t(a, b):
    return jax.lax.dot_general(
        a, b, (((1,), (1,)), ((), ())),
        precision=jax.lax.Precision.DEFAULT,
        preferred_element_type=jnp.float32)


def _bf(a):
    return a.astype(jnp.bfloat16)


def _snf_graph(d):
    sq = jnp.sum(d * d, axis=1, keepdims=True)
    g = _mm_t(d, d)
    d2 = jnp.maximum(sq + jnp.transpose(sq) - 2.0 * g, 0.0)
    dist = jnp.sqrt(d2)
    sigma = (jnp.sum(dist) / float(_N * (_N - 1))) * 0.5
    w = jnp.exp(-(dist * dist) / (2.0 * (sigma * sigma)))
    dcol = jnp.sum(w, axis=1, keepdims=True)
    dinv = 1.0 / jnp.sqrt(dcol)
    p = dinv * w * jnp.transpose(dinv)

    def strip_max(_, cur):
        m = jnp.max(cur, axis=1, keepdims=True)
        return jnp.where(cur >= m, -jnp.inf, cur)

    stripped = jax.lax.fori_loop(0, _K_NN - 1, strip_max, p)
    thr = jnp.max(stripped, axis=1, keepdims=True)
    wk = jnp.where(p >= thr, p, 0.0)
    return (wk + jnp.transpose(wk)) * 0.5


def _body(x0_ref, x1_ref, d0_ref, d1_ref,
          w10_ref, b10_ref, w20_ref, b20_ref,
          w11_ref, b11_ref, w21_ref, b21_ref,
          wfa_ref, wfb_ref, bf_ref, out_ref):
    ws0 = _snf_graph(d0_ref[:])
    ws1 = _snf_graph(d1_ref[:])
    wf = (ws0 + ws1) * 0.5
    ws0b = _bf(ws0)
    ws1b = _bf(ws1)

    def diff_body(_, wf):
        wfb = _bf(wf)
        t0 = _mm(_bf(_mm(ws0b, wfb)), ws0b)
        t1 = _mm(_bf(_mm(ws1b, wfb)), ws1b)
        wn = (t0 + t1) * 0.5
        dcol = jnp.sum(wn, axis=1, keepdims=True)
        dinv = 1.0 / jnp.sqrt(dcol)
        return dinv * wn * jnp.transpose(dinv)

    wf = jax.lax.fori_loop(0, _T_ITERS, diff_body, wf)
    wfb16 = _bf(wf)

    h0 = jax.nn.relu(_mm(wfb16, _bf(_mm(x0_ref[:], w10_ref[:]))) + b10_ref[:])
    h0 = _mm(wfb16, _bf(_mm(_bf(h0), w20_ref[:]))) + b20_ref[:]
    h1 = jax.nn.relu(_mm(wfb16, _bf(_mm(x1_ref[:], w11_ref[:]))) + b11_ref[:])
    h1 = _mm(wfb16, _bf(_mm(_bf(h1), w21_ref[:]))) + b21_ref[:]

    out_ref[:] = _mm(h0, wfa_ref[:]) + _mm(h1, wfb_ref[:]) + bf_ref[:]


def kernel(x0, x1, adj0, adj1, W1_0, b1_0, W2_0, b2_0,
           W1_1, b1_1, W2_1, b2_1, Wfuse, bfuse):
    hidden = W2_0.shape[1]
    return pl.pallas_call(
        _body,
        out_shape=jax.ShapeDtypeStruct((x0.shape[0], Wfuse.shape[1]),
                                       jnp.float32),
    )(x0, x1, adj0, adj1,
      W1_0, b1_0.reshape(1, -1), W2_0, b2_0.reshape(1, -1),
      W1_1, b1_1.reshape(1, -1), W2_1, b2_1.reshape(1, -1),
      Wfuse[:hidden], Wfuse[hidden:], bfuse.reshape(1, -1))


# speedup vs baseline: 4.3712x; 1.0158x over previous
import jax
import jax.numpy as jnp
from jax.experimental import pallas as pl

_N = 1024
_K_NN = 20
_T_ITERS = 20


def _mm(a, b):
    return jax.lax.dot_general(
        a, b, (((1,), (0,)), ((), ())),
        precision=jax.lax.Precision.DEFAULT,
        preferred_element_type=jnp.float32)


def _mm_t(a, b):
    return jax.lax.dot_general(
        a, b, (((1,), (1,)), ((), ())),
        precision=jax.lax.Precision.DEFAULT,
        preferred_element_type=jnp.float32)


def _bf(a):
    return a.astype(jnp.bfloat16)


def _snf_graph(d):
    sq = jnp.sum(d * d, axis=1, keepdims=True)
    g = _mm_t(d, d)
    d2 = jnp.maximum(sq + jnp.transpose(sq) - 2.0 * g, 0.0)
    dist = jnp.sqrt(d2)
    sigma = (jnp.sum(dist) / float(_N * (_N - 1))) * 0.5
    w = jnp.exp(-(dist * dist) / (2.0 * (sigma * sigma)))
    dcol = jnp.sum(w, axis=1, keepdims=True)
    dinv = 1.0 / jnp.sqrt(dcol)
    p = dinv * w * jnp.transpose(dinv)

    def strip_max(_, cur):
        m = jnp.max(cur, axis=1, keepdims=True)
        return jnp.where(cur >= m, -jnp.inf, cur)

    stripped = jax.lax.fori_loop(0, _K_NN - 1, strip_max, p)
    thr = jnp.max(stripped, axis=1, keepdims=True)
    wk = jnp.where(p >= thr, p, 0.0)
    return (wk + jnp.transpose(wk)) * 0.5


def _body(x0_ref, x1_ref, d0_ref, d1_ref,
          w10_ref, b10_ref, w20_ref, b20_ref,
          w11_ref, b11_ref, w21_ref, b21_ref,
          wfa_ref, wfb_ref, bf_ref, out_ref):
    ws0 = _snf_graph(d0_ref[:])
    ws1 = _snf_graph(d1_ref[:])
    wfb16 = _bf((ws0 + ws1) * 0.5)
    ws0b = _bf(ws0)
    ws1b = _bf(ws1)

    def diff_body(_, wfb):
        t0 = _mm(_bf(_mm(ws0b, wfb)), ws0b)
        t1 = _mm(_bf(_mm(ws1b, wfb)), ws1b)
        wn = (t0 + t1) * 0.5
        dcol = jnp.sum(wn, axis=1, keepdims=True)
        dinv = 1.0 / jnp.sqrt(dcol)
        return _bf(dinv * wn * jnp.transpose(dinv))

    wfb16 = jax.lax.fori_loop(0, _T_ITERS, diff_body, wfb16)

    h0 = jax.nn.relu(_mm(wfb16, _bf(_mm(x0_ref[:], w10_ref[:]))) + b10_ref[:])
    h0 = _mm(wfb16, _bf(_mm(_bf(h0), w20_ref[:]))) + b20_ref[:]
    h1 = jax.nn.relu(_mm(wfb16, _bf(_mm(x1_ref[:], w11_ref[:]))) + b11_ref[:])
    h1 = _mm(wfb16, _bf(_mm(_bf(h1), w21_ref[:]))) + b21_ref[:]

    out_ref[:] = _mm(h0, wfa_ref[:]) + _mm(h1, wfb_ref[:]) + bf_ref[:]


def kernel(x0, x1, adj0, adj1, W1_0, b1_0, W2_0, b2_0,
           W1_1, b1_1, W2_1, b2_1, Wfuse, bfuse):
    hidden = W2_0.shape[1]
    return pl.pallas_call(
        _body,
        out_shape=jax.ShapeDtypeStruct((x0.shape[0], Wfuse.shape[1]),
                                       jnp.float32),
    )(x0, x1, adj0, adj1,
      W1_0, b1_0.reshape(1, -1), W2_0, b2_0.reshape(1, -1),
      W1_1, b1_1.reshape(1, -1), W2_1, b2_1.reshape(1, -1),
      Wfuse[:hidden], Wfuse[hidden:], bfuse.reshape(1, -1))
